# Initial kernel scaffold; baseline (speedup 1.0000x reference)
import jax
import jax.numpy as jnp
from jax.experimental import pallas as pl


def kernel(x, pos, batch, mlp1, mlp2):
    raise NotImplementedError("write your pallas kernel here")


# speedup vs baseline: 7.6488x; 7.6488x over previous
import functools
import math

import jax
import jax.numpy as jnp
import numpy as np
from jax.experimental import pallas as pl
from jax.experimental.pallas import tpu as pltpu

BN = 8
PN = 1024
EPSBN = 1e-5
MAXK = 64
NEG_INF = float("-inf")


def _fps_body(n, p, posT_ref, qposT_ref):
    px = posT_ref[:, 0, :]
    py = posT_ref[:, 1, :]
    pz = posT_ref[:, 2, :]
    iota_p = jax.lax.broadcasted_iota(jnp.int32, (BN, p), 1)
    iota_n = jax.lax.broadcasted_iota(jnp.int32, (BN, n), 1)

    sx0 = px[:, 0:1]
    sy0 = py[:, 0:1]
    sz0 = pz[:, 0:1]
    d0 = (px - sx0) ** 2 + (py - sy0) ** 2 + (pz - sz0) ** 2
    first = (iota_n == 0).astype(jnp.float32)
    qx0 = sx0 * first
    qy0 = sy0 * first
    qz0 = sz0 * first

    def step(i, carry):
        d, qx, qy, qz = carry
        m = jnp.max(d, axis=1, keepdims=True)
        cand = jnp.where(d == m, iota_p, p)
        nxt = jnp.min(cand, axis=1, keepdims=True)
        oh = (iota_p == nxt).astype(jnp.float32)
        sx = jnp.sum(px * oh, axis=1, keepdims=True)
        sy = jnp.sum(py * oh, axis=1, keepdims=True)
        sz = jnp.sum(pz * oh, axis=1, keepdims=True)
        dn = (px - sx) ** 2 + (py - sy) ** 2 + (pz - sz) ** 2
        d = jnp.minimum(d, dn)
        ohn = (iota_n == i).astype(jnp.float32)
        qx = qx + sx * ohn
        qy = qy + sy * ohn
        qz = qz + sz * ohn
        return d, qx, qy, qz

    _, qx, qy, qz = jax.lax.fori_loop(1, n, step, (d0, qx0, qy0, qz0))
    qposT_ref[:, 0, :] = qx
    qposT_ref[:, 1, :] = qy
    qposT_ref[:, 2, :] = qz


def _fps(posT, n):
    p = posT.shape[-1]
    return pl.pallas_call(
        functools.partial(_fps_body, n, p),
        out_shape=jax.ShapeDtypeStruct((BN, 3, n), jnp.float32),
    )(posT)


def _thresh_body(r2, big_bits, r2_bits, posT_ref, qposT_ref, t_ref):
    px = posT_ref[0, 0, :]
    py = posT_ref[0, 1, :]
    pz = posT_ref[0, 2, :]
    qx = qposT_ref[0, 0, :]
    qy = qposT_ref[0, 1, :]
    qz = qposT_ref[0, 2, :]
    d2 = ((qx[:, None] - px[None, :]) ** 2
          + (qy[:, None] - py[None, :]) ** 2
          + (qz[:, None] - pz[None, :]) ** 2)
    score = jnp.where(d2 <= r2, d2, jnp.float32(1e9))
    bits = jax.lax.bitcast_convert_type(score, jnp.int32)

    n = qx.shape[0]
    lo0 = jnp.zeros((n,), jnp.int32)
    hi0 = jnp.full((n,), big_bits, jnp.int32)

    def step(_, carry):
        lo, hi = carry
        mid = lo + (hi - lo) // 2
        cnt = jnp.sum((bits <= mid[:, None]).astype(jnp.int32), axis=1)
        ge = cnt >= MAXK
        hi = jnp.where(ge, mid, hi)
        lo = jnp.where(ge, lo, mid + 1)
        return lo, hi

    lo, hi = jax.lax.fori_loop(0, 31, step, (lo0, hi0))
    t_bits = jnp.minimum(hi, r2_bits)
    t_ref[0, 0, :] = jax.lax.bitcast_convert_type(t_bits, jnp.float32)


def _thresholds(posT, qposT, r):
    p = posT.shape[-1]
    n = qposT.shape[-1]
    r2 = np.float32(r) * np.float32(r)
    big_bits = int(np.float32(1e9).view(np.int32))
    r2_bits = int(np.float32(r2).view(np.int32))
    return pl.pallas_call(
        functools.partial(_thresh_body, r2, big_bits, r2_bits),
        grid=(BN,),
        in_specs=[
            pl.BlockSpec((1, 3, p), lambda b: (b, 0, 0)),
            pl.BlockSpec((1, 3, n), lambda b: (b, 0, 0)),
        ],
        out_specs=pl.BlockSpec((1, 1, n), lambda b: (b, 0, 0)),
        out_shape=jax.ShapeDtypeStruct((BN, 1, n), jnp.float32),
    )(posT, qposT)


def _mlp_body(r2, qb, p, table_ref, posT_ref, qposQ_ref, tQ_ref,
              w1_ref, b1_ref, wp_ref, w2_ref, b2_ref, w3_ref, b3_ref,
              s3_ref, bt3_ref, out_ref):
    c1 = w1_ref.shape[1]
    c3 = w3_ref.shape[1]
    table = table_ref[0]
    A = jnp.dot(table, w1_ref[...]) + b1_ref[...]

    qx = qposQ_ref[0, 0, :, 0:1]
    qy = qposQ_ref[0, 0, :, 1:2]
    qz = qposQ_ref[0, 0, :, 2:3]
    Bq = (qx * wp_ref[0:1, :]
          + qy * wp_ref[1:2, :]
          + qz * wp_ref[2:3, :])

    z1 = A[None, :, :] - Bq[:, None, :]
    h1 = jnp.maximum(z1, 0.0).reshape(qb * p, c1)
    z2 = jnp.dot(h1, w2_ref[...]) + b2_ref[...]
    h2 = jnp.maximum(z2, 0.0)
    z3 = jnp.dot(h2, w3_ref[...]) + b3_ref[...]
    h3 = jnp.maximum(z3, 0.0).reshape(qb, p, c3)

    px = posT_ref[0, 0, :]
    py = posT_ref[0, 1, :]
    pz = posT_ref[0, 2, :]
    d2 = ((qx - px[None, :]) ** 2
          + (qy - py[None, :]) ** 2
          + (qz - pz[None, :]) ** 2)
    pen = jnp.where(d2 <= tQ_ref[0, 0, :, 0:1], 0.0, NEG_INF)
    hm = h3 + pen[:, :, None]
    pooled = jnp.max(hm, axis=1)
    out_ref[0] = pooled * s3_ref[...] + bt3_ref[...]


def _fold_params(params, c_in, c_in_pad):
    (W1, b1, g1, bt1), (W2, b2, g2, bt2), (W3, b3, g3, bt3) = params
    inv = np.float32(1.0) / jnp.sqrt(jnp.float32(1.0 + EPSBN))
    s1 = g1 * inv
    s2 = g2 * inv
    s3 = g3 * inv
    W1p = jnp.zeros((c_in_pad, W1.shape[1]), jnp.float32).at[:c_in].set(W1)
    Wp = W1[c_in - 3:c_in]
    W2f = s1[:, None] * W2
    b2f = (b2 + bt1 @ W2)[None, :]
    W3f = s2[:, None] * W3
    b3f = (b3 + bt2 @ W3)[None, :]
    return W1p, b1[None, :], Wp, W2f, b2f, W3f, b3f, s3[None, :], bt3[None, :]


def _mlp_pool(table, posT, qposT, t, folded, r, qb):
    p = table.shape[1]
    cpad = table.shape[2]
    n = qposT.shape[-1]
    W1p, b1, Wp, W2f, b2f, W3f, b3f, s3, bt3 = folded
    c1 = W1p.shape[1]
    c2 = W2f.shape[1]
    c3 = W3f.shape[1]
    r2 = np.float32(r) * np.float32(r)
    nqb = n // qb
    qposQ = qposT.transpose(0, 2, 1).reshape(BN, nqb, qb, 3)
    tQ = t.reshape(BN, n)[..., None].reshape(BN, nqb, qb, 1)
    full = lambda shape: pl.BlockSpec(shape, lambda b, q: tuple(0 for _ in shape))
    return pl.pallas_call(
        functools.partial(_mlp_body, r2, qb, p),
        grid=(BN, nqb),
        in_specs=[
            pl.BlockSpec((1, p, cpad), lambda b, q: (b, 0, 0)),
            pl.BlockSpec((1, 3, p), lambda b, q: (b, 0, 0)),
            pl.BlockSpec((1, 1, qb, 3), lambda b, q: (b, q, 0, 0)),
            pl.BlockSpec((1, 1, qb, 1), lambda b, q: (b, q, 0, 0)),
            full((cpad, c1)), full((1, c1)), full((3, c1)),
            full((c1, c2)), full((1, c2)),
            full((c2, c3)), full((1, c3)),
            full((1, c3)), full((1, c3)),
        ],
        out_specs=pl.BlockSpec((1, qb, c3), lambda b, q: (b, q, 0)),
        out_shape=jax.ShapeDtypeStruct((BN, n, c3), jnp.float32),
    )(table, posT, qposQ, tQ, W1p, b1, Wp, W2f, b2f, W3f, b3f, s3, bt3)


def _sa_stage(table, posT, ratio, r, folded, qb):
    p = posT.shape[-1]
    n = int(p * ratio)
    qposT = _fps(posT, n)
    t = _thresholds(posT, qposT, r)
    xo = _mlp_pool(table, posT, qposT, t, folded, r, qb)
    return xo, qposT


def kernel(x, pos, batch, mlp1, mlp2):
    xb = x.reshape(BN, PN, -1)
    posb = pos.reshape(BN, PN, 3)
    posT = posb.transpose(0, 2, 1)

    fd = xb.shape[-1]
    folded1 = _fold_params(mlp1, fd + 3, 8)
    table1 = jnp.concatenate(
        [xb, posb, jnp.zeros((BN, PN, 8 - fd - 3), jnp.float32)], axis=-1)

    x1, qposT1 = _sa_stage(table1, posT, 0.5, 0.2, folded1, 8)

    n1 = x1.shape[1]
    c_in2 = x1.shape[2] + 3
    pad2 = 136
    pos1 = qposT1.transpose(0, 2, 1)
    folded2 = _fold_params(mlp2, c_in2, pad2)
    table2 = jnp.concatenate(
        [x1, pos1, jnp.zeros((BN, n1, pad2 - c_in2), jnp.float32)], axis=-1)

    x2, qposT2 = _sa_stage(table2, qposT1, 0.25, 0.4, folded2, 16)

    n2 = x2.shape[1]
    pos2 = qposT2.transpose(0, 2, 1).reshape(BN * n2, 3)
    batch2 = jnp.repeat(jnp.arange(BN, dtype=jnp.int32), n2)
    return x2.reshape(BN * n2, -1), pos2, batch2
